# Initial kernel scaffold; baseline (speedup 1.0000x reference)
import jax
import jax.numpy as jnp
from jax.experimental import pallas as pl


def kernel(x, edge_index, W1l, b1l, W1r, g1, bt1, W2l, b2l, W2r, g2, bt2, W3l, b3l, W3r, g3, bt3, W4l, b4l, W4r, g4, bt4):
    raise NotImplementedError("write your pallas kernel here")


# speedup vs baseline: 4.2514x; 4.2514x over previous
import functools

import jax
import jax.numpy as jnp
from jax import lax
from jax.experimental import pallas as pl
from jax.experimental.pallas import tpu as pltpu
from jax.experimental.pallas import tpu_sc as plsc

_N = 10000
_E = 160000
_LANE = 128
_CHUNK = 80
_EROWS = _E // _CHUNK
_TROWS = _EROWS // 16
_NPAD = 10240
_NROWS = _NPAD // 16
_BM = 400
_GRID = _N // _BM
_CNTW = 128


def _sc_mesh():
    return plsc.VectorSubcoreMesh(core_axis_name="c", subcore_axis_name="s",
                                  num_cores=2, num_subcores=16)


@functools.cache
def _sc_agg(P):
    half = P // 2

    def body(*refs):
        xs = refs[:P]
        src_r, dst_r, zero_r = refs[P:P + 3]
        outs = refs[P + 3:2 * P + 3]
        idx_s, idx_d, rows, acc = refs[2 * P + 3:]
        c = lax.axis_index("c")
        s = lax.axis_index("s")
        pltpu.sync_copy(src_r.at[s], idx_s)
        pltpu.sync_copy(dst_r.at[s], idx_d)
        for p in range(P):
            @pl.when(c == (p // half))
            def _(p=p):
                pltpu.sync_copy(zero_r, acc.at[pl.ds(s * _NROWS, _NROWS)])
                plsc.subcore_barrier()

                @pl.loop(0, _TROWS)
                def _(i):
                    pltpu.sync_copy(xs[p].at[idx_s.at[i]], rows)
                    pltpu.sync_copy(rows, acc.at[idx_d.at[i]], add=True)

                plsc.subcore_barrier()
                pltpu.sync_copy(acc.at[pl.ds(s * _NROWS, _NROWS)],
                                outs[p].at[pl.ds(s * _NROWS, _NROWS)])

    out_type = tuple(jax.ShapeDtypeStruct((_NPAD, _LANE), jnp.float32)
                     for _ in range(P))
    return pl.kernel(
        body,
        out_type=out_type,
        mesh=_sc_mesh(),
        scratch_types=[
            pltpu.VMEM((_TROWS, _CHUNK), jnp.int32),
            pltpu.VMEM((_TROWS, _CHUNK), jnp.int32),
            pltpu.VMEM((_CHUNK, _LANE), jnp.float32),
            pltpu.VMEM_SHARED((_NPAD, _LANE), jnp.float32),
        ],
    )


@functools.cache
def _sc_cnt():
    def body(dst_r, zero_r, one_r, out_r, idx_d, ones_v, acc):
        c = lax.axis_index("c")
        s = lax.axis_index("s")

        @pl.when(c == 0)
        def _():
            pltpu.sync_copy(dst_r.at[s], idx_d)
            pltpu.sync_copy(one_r, ones_v)
            pltpu.sync_copy(zero_r, acc.at[pl.ds(s * _NROWS, _NROWS)])
            plsc.subcore_barrier()

            @pl.loop(0, _TROWS)
            def _(i):
                pltpu.sync_copy(ones_v, acc.at[idx_d.at[i]], add=True)

            plsc.subcore_barrier()
            pltpu.sync_copy(acc.at[pl.ds(s * _NROWS, _NROWS)],
                            out_r.at[pl.ds(s * _NROWS, _NROWS)])

    return pl.kernel(
        body,
        out_type=jax.ShapeDtypeStruct((_NPAD, _CNTW), jnp.float32),
        mesh=_sc_mesh(),
        scratch_types=[
            pltpu.VMEM((_TROWS, _CHUNK), jnp.int32),
            pltpu.VMEM((_CHUNK, _CNTW), jnp.float32),
            pltpu.VMEM_SHARED((_NPAD, _CNTW), jnp.float32),
        ],
    )


@functools.cache
def _t1(Din, Dout):
    P = Din // _LANE

    def body(*refs):
        aggs = refs[:P]
        hs = refs[P:2 * P]
        cnt_ref, wl_ref, wr_ref, bl_ref, y_ref, ssum_ref, ssq_ref = refs[2 * P:]
        inv = 1.0 / jnp.maximum(cnt_ref[...], 1.0)
        acc = jnp.zeros((_BM, Dout), jnp.float32)
        for p in range(P):
            sl = slice(p * _LANE, (p + 1) * _LANE)
            acc += jnp.dot(aggs[p][...] * inv, wl_ref[sl, :],
                           preferred_element_type=jnp.float32)
            acc += jnp.dot(hs[p][...], wr_ref[sl, :],
                           preferred_element_type=jnp.float32)
        y = acc + bl_ref[...]
        y_ref[...] = y

        @pl.when(pl.program_id(0) == 0)
        def _():
            ssum_ref[...] = jnp.zeros_like(ssum_ref)
            ssq_ref[...] = jnp.zeros_like(ssq_ref)

        ssum_ref[...] += jnp.sum(y, axis=0, keepdims=True)
        ssq_ref[...] += jnp.sum(y * y, axis=0, keepdims=True)

    slab = pl.BlockSpec((_BM, _LANE), lambda i: (i, 0))
    in_specs = (
        [slab] * P + [slab] * P +
        [pl.BlockSpec((_BM, 1), lambda i: (i, 0)),
         pl.BlockSpec((Din, Dout), lambda i: (0, 0)),
         pl.BlockSpec((Din, Dout), lambda i: (0, 0)),
         pl.BlockSpec((1, Dout), lambda i: (0, 0))]
    )
    out_specs = (
        pl.BlockSpec((_BM, Dout), lambda i: (i, 0)),
        pl.BlockSpec((1, Dout), lambda i: (0, 0)),
        pl.BlockSpec((1, Dout), lambda i: (0, 0)),
    )
    out_shape = (
        jax.ShapeDtypeStruct((_N, Dout), jnp.float32),
        jax.ShapeDtypeStruct((1, Dout), jnp.float32),
        jax.ShapeDtypeStruct((1, Dout), jnp.float32),
    )
    return pl.pallas_call(body, grid=(_GRID,), in_specs=in_specs,
                          out_specs=out_specs, out_shape=out_shape)


@functools.cache
def _t2(Dout, final):
    P = Dout // _LANE

    def body(y_ref, ssum_ref, ssq_ref, g_ref, bt_ref, *outs):
        nrec = 1.0 / _N
        mu = ssum_ref[...] * nrec
        var = ssq_ref[...] * nrec - mu * mu
        scale = lax.rsqrt(var + 1e-5) * g_ref[...]
        r = jnp.maximum((y_ref[...] - mu) * scale + bt_ref[...], 0.0)
        if final:
            outs[0][...] = r
        else:
            for p in range(P):
                outs[p][...] = r[:, p * _LANE:(p + 1) * _LANE]

    in_specs = [
        pl.BlockSpec((_BM, Dout), lambda i: (i, 0)),
        pl.BlockSpec((1, Dout), lambda i: (0, 0)),
        pl.BlockSpec((1, Dout), lambda i: (0, 0)),
        pl.BlockSpec((1, Dout), lambda i: (0, 0)),
        pl.BlockSpec((1, Dout), lambda i: (0, 0)),
    ]
    if final:
        out_specs = pl.BlockSpec((_BM, Dout), lambda i: (i, 0))
        out_shape = jax.ShapeDtypeStruct((_N, Dout), jnp.float32)
    else:
        out_specs = tuple(pl.BlockSpec((_BM, _LANE), lambda i: (i, 0))
                          for _ in range(P))
        out_shape = tuple(jax.ShapeDtypeStruct((_N, _LANE), jnp.float32)
                          for _ in range(P))
    return pl.pallas_call(body, grid=(_GRID,), in_specs=in_specs,
                          out_specs=out_specs, out_shape=out_shape)


def kernel(x, edge_index, W1l, b1l, W1r, g1, bt1, W2l, b2l, W2r, g2, bt2,
           W3l, b3l, W3r, g3, bt3, W4l, b4l, W4r, g4, bt4):
    src2d = edge_index[0].reshape(16, _TROWS, _CHUNK)
    dst2d = edge_index[1].reshape(16, _TROWS, _CHUNK)
    zeros = jnp.zeros((_NROWS, _LANE), jnp.float32)
    zeros_c = jnp.zeros((_NROWS, _CNTW), jnp.float32)
    ones_c = jnp.ones((_CHUNK, _CNTW), jnp.float32)

    cnt16 = _sc_cnt()(dst2d, zeros_c, ones_c)
    cnt = cnt16[:, 0:1]

    h = [x[:, p * _LANE:(p + 1) * _LANE] for p in range(x.shape[1] // _LANE)]
    layers = [
        (W1l, b1l, W1r, g1, bt1),
        (W2l, b2l, W2r, g2, bt2),
        (W3l, b3l, W3r, g3, bt3),
        (W4l, b4l, W4r, g4, bt4),
    ]
    for li, (Wl, bl, Wr, g, bt) in enumerate(layers):
        Din, Dout = Wl.shape
        P = Din // _LANE
        aggs = _sc_agg(P)(*h, src2d, dst2d, zeros)
        y, ssum, ssq = _t1(Din, Dout)(*aggs, *h, cnt, Wl, Wr,
                                      bl.reshape(1, Dout))
        final = li == len(layers) - 1
        out = _t2(Dout, final)(y, ssum, ssq, g.reshape(1, Dout),
                               bt.reshape(1, Dout))
        h = [out] if final else list(out)
    return h[0]
